# Initial kernel scaffold; baseline (speedup 1.0000x reference)
import jax
import jax.numpy as jnp
from jax.experimental import pallas as pl


def kernel(node_feat, node_depth, edge_index, edge_attr, batch, type_emb, attr_emb, depth_emb, vn_emb, eps, edge_W, edge_b, W1, b1, bn1_g, bn1_b, W2, b2, bn_g, bn_b, vnW1, vnb1, vnbn1_g, vnbn1_b, vnW2, vnb2, vnbn2_g, vnbn2_b, predW, predb):
    raise NotImplementedError("write your pallas kernel here")


# speedup vs baseline: 1.0084x; 1.0084x over previous
import functools

import jax
import jax.numpy as jnp
from jax.experimental import pallas as pl
from jax.experimental.pallas import tpu as pltpu

D = 100
L = 5
MSL = 5
V = 5002
G = 128
MAX_DEPTH = 20
N = 50000
E = 800000

_NB = 25
_BR = N // _NB


def _mm_stats_body(x_ref, w_ref, b_ref, z_ref, s_ref, q_ref):
    i = pl.program_id(0)
    z = jnp.dot(x_ref[...], w_ref[...], preferred_element_type=jnp.float32)
    z = z + b_ref[...]
    z_ref[...] = z

    @pl.when(i == 0)
    def _():
        s_ref[...] = jnp.zeros_like(s_ref)
        q_ref[...] = jnp.zeros_like(q_ref)

    s_ref[...] += jnp.sum(z, axis=0, keepdims=True)
    q_ref[...] += jnp.sum(z * z, axis=0, keepdims=True)


def _mm_stats(x, w, b):
    n, k = x.shape
    f = w.shape[1]
    nb = n // _BR
    return pl.pallas_call(
        _mm_stats_body,
        grid=(nb,),
        in_specs=[
            pl.BlockSpec((_BR, k), lambda i: (i, 0)),
            pl.BlockSpec((k, f), lambda i: (0, 0)),
            pl.BlockSpec((1, f), lambda i: (0, 0)),
        ],
        out_specs=[
            pl.BlockSpec((_BR, f), lambda i: (i, 0)),
            pl.BlockSpec((1, f), lambda i: (0, 0)),
            pl.BlockSpec((1, f), lambda i: (0, 0)),
        ],
        out_shape=[
            jax.ShapeDtypeStruct((n, f), jnp.float32),
            jax.ShapeDtypeStruct((1, f), jnp.float32),
            jax.ShapeDtypeStruct((1, f), jnp.float32),
        ],
    )(x, w, b.reshape(1, f))


def _bn_relu_mm_stats_body(z1_ref, s1_ref, q1_ref, g_ref, be_ref, w_ref,
                           b_ref, z_ref, s_ref, q_ref):
    i = pl.program_id(0)
    m = s1_ref[...] / N
    var = q1_ref[...] / N - m * m
    scale = g_ref[...] * jax.lax.rsqrt(var + 1e-5)
    shift = be_ref[...] - m * scale
    a = jnp.maximum(z1_ref[...] * scale + shift, 0.0)
    z = jnp.dot(a, w_ref[...], preferred_element_type=jnp.float32)
    z = z + b_ref[...]
    z_ref[...] = z

    @pl.when(i == 0)
    def _():
        s_ref[...] = jnp.zeros_like(s_ref)
        q_ref[...] = jnp.zeros_like(q_ref)

    s_ref[...] += jnp.sum(z, axis=0, keepdims=True)
    q_ref[...] += jnp.sum(z * z, axis=0, keepdims=True)


def _bn_relu_mm_stats(z1, s1, q1, g, be, w, b):
    n, f1 = z1.shape
    f = w.shape[1]
    nb = n // _BR
    return pl.pallas_call(
        _bn_relu_mm_stats_body,
        grid=(nb,),
        in_specs=[
            pl.BlockSpec((_BR, f1), lambda i: (i, 0)),
            pl.BlockSpec((1, f1), lambda i: (0, 0)),
            pl.BlockSpec((1, f1), lambda i: (0, 0)),
            pl.BlockSpec((1, f1), lambda i: (0, 0)),
            pl.BlockSpec((1, f1), lambda i: (0, 0)),
            pl.BlockSpec((f1, f), lambda i: (0, 0)),
            pl.BlockSpec((1, f), lambda i: (0, 0)),
        ],
        out_specs=[
            pl.BlockSpec((_BR, f), lambda i: (i, 0)),
            pl.BlockSpec((1, f), lambda i: (0, 0)),
            pl.BlockSpec((1, f), lambda i: (0, 0)),
        ],
        out_shape=[
            jax.ShapeDtypeStruct((n, f), jnp.float32),
            jax.ShapeDtypeStruct((1, f), jnp.float32),
            jax.ShapeDtypeStruct((1, f), jnp.float32),
        ],
    )(z1, s1, q1, g.reshape(1, f1), be.reshape(1, f1), w, b.reshape(1, f))


def _bn_final_body(relu, z_ref, s_ref, q_ref, g_ref, be_ref, o_ref):
    m = s_ref[...] / N
    var = q_ref[...] / N - m * m
    scale = g_ref[...] * jax.lax.rsqrt(var + 1e-5)
    shift = be_ref[...] - m * scale
    o = z_ref[...] * scale + shift
    if relu:
        o = jnp.maximum(o, 0.0)
    o_ref[...] = o


def _bn_final(z, s, q, g, be, relu):
    n, f = z.shape
    nb = n // _BR
    return pl.pallas_call(
        functools.partial(_bn_final_body, relu),
        grid=(nb,),
        in_specs=[
            pl.BlockSpec((_BR, f), lambda i: (i, 0)),
            pl.BlockSpec((1, f), lambda i: (0, 0)),
            pl.BlockSpec((1, f), lambda i: (0, 0)),
            pl.BlockSpec((1, f), lambda i: (0, 0)),
            pl.BlockSpec((1, f), lambda i: (0, 0)),
        ],
        out_specs=pl.BlockSpec((_BR, f), lambda i: (i, 0)),
        out_shape=jax.ShapeDtypeStruct((n, f), jnp.float32),
    )(z, s, q, g.reshape(1, f), be.reshape(1, f))


def _vn_mlp_body(x_ref, w1_ref, b1_ref, g1_ref, be1_ref, w2_ref, b2_ref,
                 g2_ref, be2_ref, o_ref):
    x = x_ref[...]
    z1 = jnp.dot(x, w1_ref[...], preferred_element_type=jnp.float32) + b1_ref[...]
    m1 = jnp.mean(z1, axis=0, keepdims=True)
    v1 = jnp.mean(z1 * z1, axis=0, keepdims=True) - m1 * m1
    t = jnp.maximum(g1_ref[...] * (z1 - m1) * jax.lax.rsqrt(v1 + 1e-5)
                    + be1_ref[...], 0.0)
    z2 = jnp.dot(t, w2_ref[...], preferred_element_type=jnp.float32) + b2_ref[...]
    m2 = jnp.mean(z2, axis=0, keepdims=True)
    v2 = jnp.mean(z2 * z2, axis=0, keepdims=True) - m2 * m2
    o_ref[...] = jnp.maximum(g2_ref[...] * (z2 - m2) * jax.lax.rsqrt(v2 + 1e-5)
                             + be2_ref[...], 0.0)


def _vn_mlp(x, w1, b1, g1, be1, w2, b2, g2, be2):
    f1 = w1.shape[1]
    return pl.pallas_call(
        _vn_mlp_body,
        out_shape=jax.ShapeDtypeStruct((G, D), jnp.float32),
    )(x, w1, b1.reshape(1, f1), g1.reshape(1, f1), be1.reshape(1, f1),
      w2, b2.reshape(1, D), g2.reshape(1, D), be2.reshape(1, D))


def _heads_body(hg_ref, w_ref, b_ref, o_ref):
    o_ref[...] = (jnp.dot(hg_ref[...], w_ref[0],
                          preferred_element_type=jnp.float32) + b_ref[0])[None]


def _heads(h_graph, predW, predb):
    out = pl.pallas_call(
        _heads_body,
        grid=(MSL,),
        in_specs=[
            pl.BlockSpec((G, D), lambda i: (0, 0)),
            pl.BlockSpec((1, D, V), lambda i: (i, 0, 0)),
            pl.BlockSpec((1, 1, V), lambda i: (i, 0, 0)),
        ],
        out_specs=pl.BlockSpec((1, G, V), lambda i: (i, 0, 0)),
        out_shape=jax.ShapeDtypeStruct((MSL, G, V), jnp.float32),
    )(h_graph, predW, predb.reshape(MSL, 1, V))
    return tuple(out[i] for i in range(MSL))


def kernel(node_feat, node_depth, edge_index, edge_attr, batch, type_emb,
           attr_emb, depth_emb, vn_emb, eps, edge_W, edge_b, W1, b1, bn1_g,
           bn1_b, W2, b2, bn_g, bn_b, vnW1, vnb1, vnbn1_g, vnbn1_b, vnW2,
           vnb2, vnbn2_g, vnbn2_b, predW, predb):
    depth = jnp.minimum(node_depth, MAX_DEPTH)
    h = type_emb[node_feat[:, 0]] + attr_emb[node_feat[:, 1]] + depth_emb[depth]
    vn = jnp.broadcast_to(vn_emb[0], (G, D))
    src, dst = edge_index[0], edge_index[1]

    for l in range(L):
        hl = h + vn[batch]
        edge_emb = edge_attr @ edge_W[l] + edge_b[l]
        msg = jax.nn.relu(hl[src] + edge_emb)
        aggr = jax.ops.segment_sum(msg, dst, num_segments=N)
        x = (1.0 + eps[l]) * hl + aggr

        z1, s1, q1 = _mm_stats(x, W1[l], b1[l])
        z2, s2, q2 = _bn_relu_mm_stats(z1, s1, q1, bn1_g[l], bn1_b[l],
                                       W2[l], b2[l])
        h = _bn_final(z2, s2, q2, bn_g[l], bn_b[l], relu=(l < L - 1))

        if l < L - 1:
            vn_tmp = jax.ops.segment_sum(hl, batch, num_segments=G) + vn
            vn = _vn_mlp(vn_tmp, vnW1[l], vnb1[l], vnbn1_g[l], vnbn1_b[l],
                         vnW2[l], vnb2[l], vnbn2_g[l], vnbn2_b[l])

    counts = jax.ops.segment_sum(jnp.ones((N,), jnp.float32), batch,
                                 num_segments=G)
    h_graph = (jax.ops.segment_sum(h, batch, num_segments=G)
               / jnp.maximum(counts, 1.0)[:, None])
    return _heads(h_graph, predW, predb)


# speedup vs baseline: 1.0571x; 1.0482x over previous
import functools

import jax
import jax.numpy as jnp
from jax import lax
from jax.experimental import pallas as pl
from jax.experimental.pallas import tpu as pltpu

D = 100
L = 5
MSL = 5
V = 5002
G = 128
MAX_DEPTH = 20
N = 50000
E = 800000

_NB = 25
_BR = N // _NB

DP = 128


def _mm_stats_body(hl_ref, ag_ref, e_ref, w_ref, b_ref, z_ref, s_ref, q_ref):
    i = pl.program_id(0)
    x = (1.0 + e_ref[0, 0]) * hl_ref[...] + ag_ref[...]
    z = jnp.dot(x, w_ref[...], preferred_element_type=jnp.float32)
    z = z + b_ref[...]
    z_ref[...] = z

    @pl.when(i == 0)
    def _():
        s_ref[...] = jnp.zeros_like(s_ref)
        q_ref[...] = jnp.zeros_like(q_ref)

    s_ref[...] += jnp.sum(z, axis=0, keepdims=True)
    q_ref[...] += jnp.sum(z * z, axis=0, keepdims=True)


def _mm_stats(hl, aggr, eps_l, w, b):
    n, k = hl.shape
    f = w.shape[1]
    nb = n // _BR
    return pl.pallas_call(
        _mm_stats_body,
        grid=(nb,),
        in_specs=[
            pl.BlockSpec((_BR, k), lambda i: (i, 0)),
            pl.BlockSpec((_BR, k), lambda i: (i, 0)),
            pl.BlockSpec((1, 1), lambda i: (0, 0)),
            pl.BlockSpec((k, f), lambda i: (0, 0)),
            pl.BlockSpec((1, f), lambda i: (0, 0)),
        ],
        out_specs=[
            pl.BlockSpec((_BR, f), lambda i: (i, 0)),
            pl.BlockSpec((1, f), lambda i: (0, 0)),
            pl.BlockSpec((1, f), lambda i: (0, 0)),
        ],
        out_shape=[
            jax.ShapeDtypeStruct((n, f), jnp.float32),
            jax.ShapeDtypeStruct((1, f), jnp.float32),
            jax.ShapeDtypeStruct((1, f), jnp.float32),
        ],
    )(hl, aggr, eps_l.reshape(1, 1), w, b.reshape(1, f))


def _bn_relu_mm_stats_body(z1_ref, s1_ref, q1_ref, g_ref, be_ref, w_ref,
                           b_ref, z_ref, s_ref, q_ref):
    i = pl.program_id(0)
    m = s1_ref[...] / N
    var = q1_ref[...] / N - m * m
    scale = g_ref[...] * jax.lax.rsqrt(var + 1e-5)
    shift = be_ref[...] - m * scale
    a = jnp.maximum(z1_ref[...] * scale + shift, 0.0)
    z = jnp.dot(a, w_ref[...], preferred_element_type=jnp.float32)
    z = z + b_ref[...]
    z_ref[...] = z

    @pl.when(i == 0)
    def _():
        s_ref[...] = jnp.zeros_like(s_ref)
        q_ref[...] = jnp.zeros_like(q_ref)

    s_ref[...] += jnp.sum(z, axis=0, keepdims=True)
    q_ref[...] += jnp.sum(z * z, axis=0, keepdims=True)


def _bn_relu_mm_stats(z1, s1, q1, g, be, w, b):
    n, f1 = z1.shape
    f = w.shape[1]
    nb = n // _BR
    return pl.pallas_call(
        _bn_relu_mm_stats_body,
        grid=(nb,),
        in_specs=[
            pl.BlockSpec((_BR, f1), lambda i: (i, 0)),
            pl.BlockSpec((1, f1), lambda i: (0, 0)),
            pl.BlockSpec((1, f1), lambda i: (0, 0)),
            pl.BlockSpec((1, f1), lambda i: (0, 0)),
            pl.BlockSpec((1, f1), lambda i: (0, 0)),
            pl.BlockSpec((f1, f), lambda i: (0, 0)),
            pl.BlockSpec((1, f), lambda i: (0, 0)),
        ],
        out_specs=[
            pl.BlockSpec((_BR, f), lambda i: (i, 0)),
            pl.BlockSpec((1, f), lambda i: (0, 0)),
            pl.BlockSpec((1, f), lambda i: (0, 0)),
        ],
        out_shape=[
            jax.ShapeDtypeStruct((n, f), jnp.float32),
            jax.ShapeDtypeStruct((1, f), jnp.float32),
            jax.ShapeDtypeStruct((1, f), jnp.float32),
        ],
    )(z1, s1, q1, g.reshape(1, f1), be.reshape(1, f1), w, b.reshape(1, f))


def _bn_final_body(relu, z_ref, s_ref, q_ref, g_ref, be_ref, o_ref):
    m = s_ref[...] / N
    var = q_ref[...] / N - m * m
    scale = g_ref[...] * jax.lax.rsqrt(var + 1e-5)
    shift = be_ref[...] - m * scale
    o = z_ref[...] * scale + shift
    if relu:
        o = jnp.maximum(o, 0.0)
    o_ref[...] = o


def _bn_final(z, s, q, g, be, relu):
    n, f = z.shape
    nb = n // _BR
    return pl.pallas_call(
        functools.partial(_bn_final_body, relu),
        grid=(nb,),
        in_specs=[
            pl.BlockSpec((_BR, f), lambda i: (i, 0)),
            pl.BlockSpec((1, f), lambda i: (0, 0)),
            pl.BlockSpec((1, f), lambda i: (0, 0)),
            pl.BlockSpec((1, f), lambda i: (0, 0)),
            pl.BlockSpec((1, f), lambda i: (0, 0)),
        ],
        out_specs=pl.BlockSpec((_BR, f), lambda i: (i, 0)),
        out_shape=jax.ShapeDtypeStruct((n, f), jnp.float32),
    )(z, s, q, g.reshape(1, f), be.reshape(1, f))


def _vn_mlp_body(x_ref, w1_ref, b1_ref, g1_ref, be1_ref, w2_ref, b2_ref,
                 g2_ref, be2_ref, o_ref):
    x = x_ref[...]
    z1 = jnp.dot(x, w1_ref[...], preferred_element_type=jnp.float32) + b1_ref[...]
    m1 = jnp.mean(z1, axis=0, keepdims=True)
    v1 = jnp.mean(z1 * z1, axis=0, keepdims=True) - m1 * m1
    t = jnp.maximum(g1_ref[...] * (z1 - m1) * jax.lax.rsqrt(v1 + 1e-5)
                    + be1_ref[...], 0.0)
    z2 = jnp.dot(t, w2_ref[...], preferred_element_type=jnp.float32) + b2_ref[...]
    m2 = jnp.mean(z2, axis=0, keepdims=True)
    v2 = jnp.mean(z2 * z2, axis=0, keepdims=True) - m2 * m2
    o_ref[...] = jnp.maximum(g2_ref[...] * (z2 - m2) * jax.lax.rsqrt(v2 + 1e-5)
                             + be2_ref[...], 0.0)


def _vn_mlp(x, w1, b1, g1, be1, w2, b2, g2, be2):
    f1 = w1.shape[1]
    f2 = w2.shape[1]
    return pl.pallas_call(
        _vn_mlp_body,
        out_shape=jax.ShapeDtypeStruct((G, f2), jnp.float32),
    )(x, w1, b1.reshape(1, f1), g1.reshape(1, f1), be1.reshape(1, f1),
      w2, b2.reshape(1, f2), g2.reshape(1, f2), be2.reshape(1, f2))


def _heads_body(hg_ref, w_ref, b_ref, o_ref):
    o_ref[...] = (jnp.dot(hg_ref[...], w_ref[0],
                          preferred_element_type=jnp.float32) + b_ref[0])[None]


def _heads(h_graph, predW, predb):
    out = pl.pallas_call(
        _heads_body,
        grid=(MSL,),
        in_specs=[
            pl.BlockSpec((G, DP), lambda i: (0, 0)),
            pl.BlockSpec((1, DP, V), lambda i: (i, 0, 0)),
            pl.BlockSpec((1, 1, V), lambda i: (i, 0, 0)),
        ],
        out_specs=pl.BlockSpec((1, G, V), lambda i: (i, 0, 0)),
        out_shape=jax.ShapeDtypeStruct((MSL, G, V), jnp.float32),
    )(h_graph, predW, predb.reshape(MSL, 1, V))
    return tuple(out[i] for i in range(MSL))


def kernel(node_feat, node_depth, edge_index, edge_attr, batch, type_emb,
           attr_emb, depth_emb, vn_emb, eps, edge_W, edge_b, W1, b1, bn1_g,
           bn1_b, W2, b2, bn_g, bn_b, vnW1, vnb1, vnbn1_g, vnbn1_b, vnW2,
           vnb2, vnbn2_g, vnbn2_b, predW, predb):
    pad = DP - D
    depth = jnp.minimum(node_depth, MAX_DEPTH)
    h = type_emb[node_feat[:, 0]] + attr_emb[node_feat[:, 1]] + depth_emb[depth]
    h = jnp.pad(h, ((0, 0), (0, pad)))
    vn = jnp.pad(jnp.broadcast_to(vn_emb[0], (G, D)), ((0, 0), (0, pad)))

    src0 = edge_index[0].astype(jnp.int32)
    dst0 = edge_index[1].astype(jnp.int32)

    eWp = jnp.pad(edge_W, ((0, 0), (0, 0), (0, pad)))
    ebp = jnp.pad(edge_b, ((0, 0), (0, pad)))
    W1p = jnp.pad(W1, ((0, 0), (0, pad), (0, 0)))
    W2p = jnp.pad(W2, ((0, 0), (0, 0), (0, pad)))
    b2p = jnp.pad(b2, ((0, 0), (0, pad)))
    bngp = jnp.pad(bn_g, ((0, 0), (0, pad)))
    bnbp = jnp.pad(bn_b, ((0, 0), (0, pad)))
    vnW1p = jnp.pad(vnW1, ((0, 0), (0, pad), (0, 0)))
    vnW2p = jnp.pad(vnW2, ((0, 0), (0, 0), (0, pad)))
    vnb2p = jnp.pad(vnb2, ((0, 0), (0, pad)))
    vng2p = jnp.pad(vnbn2_g, ((0, 0), (0, pad)))
    vnb2bp = jnp.pad(vnbn2_b, ((0, 0), (0, pad)))
    predWp = jnp.pad(predW, ((0, 0), (0, pad), (0, 0)))

    for l in range(L):
        hl = h + vn[batch]
        msg = jax.nn.relu(hl[src0] + edge_attr @ eWp[l] + ebp[l])
        aggr = jax.ops.segment_sum(msg, dst0, num_segments=N)
        z1, s1, q1 = _mm_stats(hl, aggr, eps[l], W1p[l], b1[l])
        z2, s2, q2 = _bn_relu_mm_stats(z1, s1, q1, bn1_g[l], bn1_b[l],
                                       W2p[l], b2p[l])
        h = _bn_final(z2, s2, q2, bngp[l], bnbp[l], relu=(l < L - 1))

        if l < L - 1:
            vn_tmp = jax.ops.segment_sum(hl, batch, num_segments=G) + vn
            vn = _vn_mlp(vn_tmp, vnW1p[l], vnb1[l], vnbn1_g[l], vnbn1_b[l],
                         vnW2p[l], vnb2p[l], vng2p[l], vnb2bp[l])

    cnts2 = jax.ops.segment_sum(jnp.ones((N,), jnp.float32), batch,
                                 num_segments=G)
    h_graph = (jax.ops.segment_sum(h, batch, num_segments=G)
               / jnp.maximum(cnts2, 1.0)[:, None])
    return _heads(h_graph, predWp, predb)
